# Initial kernel scaffold; baseline (speedup 1.0000x reference)
import jax
import jax.numpy as jnp
from jax.experimental import pallas as pl


def kernel(x, edge_index, batch, gin_W1, gin_b1, gin_bn1_g, gin_bn1_b, gin_W2, gin_b2, gin_eps, bn_g, bn_b, pW1, pb1, pbn_g, pbn_b, pW2, pb2):
    raise NotImplementedError("write your pallas kernel here")


# speedup vs baseline: 4.4821x; 4.4821x over previous
import functools

import jax
import jax.numpy as jnp
from jax import lax
from jax.experimental import pallas as pl
from jax.experimental.pallas import tpu as pltpu
from jax.experimental.pallas import tpu_sc as plsc

_NEG_INF = float("-inf")


@functools.lru_cache(maxsize=None)
def _make_sc_agg(N, D, E):
    info = plsc.get_sparse_core_info()
    NC, NS = info.num_cores, info.num_subcores
    NW = NC * NS
    EC = E // NW
    CH = 80
    NCH = EC // CH
    NP = -(-N // (NS * 8)) * (NS * 8)
    RT = NP // NS
    assert EC * NW == E and CH * NCH == EC

    mesh = plsc.VectorSubcoreMesh(core_axis_name="c", subcore_axis_name="s")

    @functools.partial(
        pl.kernel,
        out_type=jax.ShapeDtypeStruct((NC, NP, D), jnp.float32),
        mesh=mesh,
        scratch_types=[
            pltpu.VMEM((CH,), jnp.int32),
            pltpu.VMEM((CH,), jnp.int32),
            pltpu.VMEM((CH, D), jnp.float32),
            pltpu.VMEM_SHARED((NP, D), jnp.float32),
            pltpu.SemaphoreType.DMA,
        ],
    )
    def sc_agg(hr, srcv, dstv, zeros, out, idxs, idxd, rows, agg, sem):
        c = lax.axis_index("c")
        s = lax.axis_index("s")
        pltpu.sync_copy(zeros, agg.at[pl.ds(s * RT, RT)])
        plsc.subcore_barrier()
        ebase = (c * NS + s) * EC

        def chunk(j, carry):
            off = pl.multiple_of(ebase + j * CH, 8)
            pltpu.sync_copy(srcv.at[pl.ds(off, CH)], idxs)
            pltpu.sync_copy(dstv.at[pl.ds(off, CH)], idxd)
            pltpu.async_copy(hr.at[idxs], rows, sem).wait()
            pltpu.sync_copy(rows, agg.at[idxd], add=True)
            return carry

        lax.fori_loop(0, NCH, chunk, 0)
        plsc.subcore_barrier()
        pltpu.sync_copy(agg.at[pl.ds(s * RT, RT)],
                        out.at[c].at[pl.ds(s * RT, RT)])

    return sc_agg


def _sc_message_agg(hr, src, dst, zeros):
    N, D = hr.shape
    E = src.shape[0]
    return _make_sc_agg(N, D, E)(hr, src, dst, zeros)


def _relu_body(x_ref, o_ref):
    o_ref[...] = jnp.maximum(x_ref[...], 0.0)


def _relu_rows(x, rb):
    N, D = x.shape
    return pl.pallas_call(
        _relu_body,
        grid=(N // rb,),
        in_specs=[pl.BlockSpec((rb, D), lambda i: (i, 0))],
        out_specs=pl.BlockSpec((rb, D), lambda i: (i, 0)),
        out_shape=jax.ShapeDtypeStruct((N, D), jnp.float32),
    )(x)


def _mlp_body(eps_ref, h_ref, agg_ref, W1_ref, b1_ref, g1_ref, B1_ref,
              W2_ref, b2_ref, g2_ref, B2_ref, hn_ref, hr_ref, *, inner_relu):
    h = h_ref[...]
    z = (1.0 + eps_ref[0]) * h + agg_ref[0] + agg_ref[1]
    a = jnp.dot(z, W1_ref[...], preferred_element_type=jnp.float32)
    a = (a + b1_ref[...]) * g1_ref[...] + B1_ref[...]
    a = jnp.maximum(a, 0.0)
    b = jnp.dot(a, W2_ref[...], preferred_element_type=jnp.float32)
    b = (b + b2_ref[...]) * g2_ref[...] + B2_ref[...]
    if inner_relu:
        b = jnp.maximum(b, 0.0)
    hn = b + h
    hn_ref[...] = hn
    hr_ref[...] = jnp.maximum(hn, 0.0)


@functools.lru_cache(maxsize=None)
def _make_mlp(N, D, rb, inner_relu):
    H = 2 * D
    grid = (N // rb,)
    row_spec = pl.BlockSpec((rb, D), lambda i: (i, 0))
    full = lambda shape: pl.BlockSpec(shape, lambda i: tuple(0 for _ in shape))
    return pl.pallas_call(
        functools.partial(_mlp_body, inner_relu=inner_relu),
        grid=grid,
        in_specs=[
            pl.BlockSpec(memory_space=pltpu.SMEM),
            row_spec,
            pl.BlockSpec((2, rb, D), lambda i: (0, i, 0)),
            full((D, H)), full((1, H)), full((1, H)), full((1, H)),
            full((H, D)), full((1, D)), full((1, D)), full((1, D)),
        ],
        out_specs=[row_spec, row_spec],
        out_shape=[jax.ShapeDtypeStruct((N, D), jnp.float32),
                   jax.ShapeDtypeStruct((N, D), jnp.float32)],
    )


def _pool_body(glo_ref, ghi_ref, h_ref, bat_ref, pW1_ref, pb1_ref, pg_ref,
               pB_ref, pW2_ref, pb2_ref, out_ref, hg_ref, acc_ref, *, nb, G):
    i = pl.program_id(0)

    @pl.when(i == 0)
    def _init():
        acc_ref[...] = jnp.full_like(acc_ref, _NEG_INF)

    blk = h_ref[...]
    bat = bat_ref[...]

    def seg(g, carry):
        m = bat == g
        v = jnp.where(m, blk, _NEG_INF)
        mx = jnp.max(v, axis=0, keepdims=True)
        acc_ref[pl.ds(g, 1), :] = jnp.maximum(acc_ref[pl.ds(g, 1), :], mx)
        return carry

    lax.fori_loop(glo_ref[i], ghi_ref[i] + 1, seg, 0)

    @pl.when(i == nb - 1)
    def _finish():
        hg = acc_ref[...]
        hg = jnp.where(hg == _NEG_INF, 0.0, hg)
        o = jnp.dot(hg, pW1_ref[...], preferred_element_type=jnp.float32)
        o = (o + pb1_ref[...]) * pg_ref[...] + pB_ref[...]
        o = jnp.maximum(o, 0.0)
        o = jnp.dot(o, pW2_ref[...], preferred_element_type=jnp.float32)
        out_ref[...] = o + pb2_ref[...]
        hg_ref[...] = hg


@functools.lru_cache(maxsize=None)
def _make_pool(N, D, G, T, rb):
    H = 2 * D
    nb = N // rb
    full = lambda shape: pl.BlockSpec(shape, lambda i: tuple(0 for _ in shape))
    return pl.pallas_call(
        functools.partial(_pool_body, nb=nb, G=G),
        grid=(nb,),
        in_specs=[
            pl.BlockSpec(memory_space=pltpu.SMEM),
            pl.BlockSpec(memory_space=pltpu.SMEM),
            pl.BlockSpec((rb, D), lambda i: (i, 0)),
            pl.BlockSpec((rb, 1), lambda i: (i, 0)),
            full((D, H)), full((1, H)), full((1, H)), full((1, H)),
            full((H, T)), full((1, T)),
        ],
        out_specs=[full((G, T)), full((G, D))],
        out_shape=[jax.ShapeDtypeStruct((G, T), jnp.float32),
                   jax.ShapeDtypeStruct((G, D), jnp.float32)],
        scratch_shapes=[pltpu.VMEM((G, D), jnp.float32)],
    )


def kernel(x, edge_index, batch, gin_W1, gin_b1, gin_bn1_g, gin_bn1_b,
           gin_W2, gin_b2, gin_eps, bn_g, bn_b,
           pW1, pb1, pbn_g, pbn_b, pW2, pb2):
    N, D = x.shape
    E = edge_index.shape[1]
    L = gin_W1.shape[0]
    G = 256
    T = pW2.shape[1]
    RB = 2000

    src = edge_index[0]
    dst = edge_index[1]
    info = plsc.get_sparse_core_info()
    NS = info.num_subcores
    NP = -(-N // (NS * 8)) * (NS * 8)
    zeros = jnp.zeros((NP // NS, D), jnp.float32)

    r2 = lambda v: v.reshape(1, -1)

    h = x
    hr = _relu_rows(x, RB)
    for l in range(L):
        agg2 = _sc_message_agg(hr, src, dst, zeros)
        mlp = _make_mlp(N, D, RB, inner_relu=(l < L - 1))
        h, hr = mlp(gin_eps[l].reshape(1), h, agg2,
                    gin_W1[l], r2(gin_b1[l]), r2(gin_bn1_g[l]), r2(gin_bn1_b[l]),
                    gin_W2[l], r2(gin_b2[l]), r2(bn_g[l]), r2(bn_b[l]))

    glo = batch[::RB].astype(jnp.int32)
    ghi = batch[RB - 1::RB].astype(jnp.int32)
    bat2 = batch.astype(jnp.int32).reshape(N, 1)

    pool = _make_pool(N, D, G, T, RB)
    out, hg = pool(glo, ghi, h, bat2, pW1, r2(pb1), r2(pbn_g), r2(pbn_b),
                   pW2, r2(pb2))
    return (out, hg)
